# baseline (speedup 1.0000x reference)
import functools
import math

import jax
import jax.numpy as jnp
from jax.experimental import pallas as pl
from jax.experimental.pallas import tpu as pltpu


def _attn_tile_kernel(q_ref, k_ref, v_ref, o_ref, acc_ref, *, scale, block_k,
                      num_tiles):
    t = pl.program_id(1)
    q = (q_ref[0] * (scale * 1.4426950408889634)).astype(jnp.bfloat16)
    tile = k_ref.shape[1]
    d = q_ref.shape[2]
    ps = []
    for j in range(tile // block_k):
        kj = k_ref[0, pl.ds(j * block_k, block_k), :].astype(jnp.bfloat16)
        s = jax.lax.dot_general(
            q, kj, (((1,), (1,)), ((), ())), preferred_element_type=jnp.float32
        )
        ps.append(jnp.exp2(s).astype(jnp.bfloat16))
    pt = jnp.concatenate(ps, axis=1)
    va = jnp.concatenate(
        [v_ref[0].astype(jnp.bfloat16), jnp.ones((tile, 128), jnp.bfloat16)],
        axis=1,
    )
    ot = jax.lax.dot_general(
        pt, va, (((1,), (0,)), ((), ())), preferred_element_type=jnp.float32
    )

    @pl.when(t == 0)
    def _init():
        acc_ref[...] = ot

    @pl.when(t > 0)
    def _accum():
        acc_ref[...] += ot

    @pl.when(t == num_tiles - 1)
    def _finish():
        acc = acc_ref[...]
        o_ref[0] = acc[:, :d] / acc[:, d : d + 1]


def kernel(q, k, v):
    B, Lq, d = q.shape
    L = k.shape[1]
    block_k = 128
    num_tiles = 2
    tile = L // num_tiles
    scale = 1.0 / math.sqrt(d)
    return pl.pallas_call(
        functools.partial(
            _attn_tile_kernel, scale=scale, block_k=block_k, num_tiles=num_tiles
        ),
        grid=(B, num_tiles),
        in_specs=[
            pl.BlockSpec((1, Lq, d), lambda b, t: (b, 0, 0)),
            pl.BlockSpec((1, tile, d), lambda b, t: (b, t, 0)),
            pl.BlockSpec((1, tile, d), lambda b, t: (b, t, 0)),
        ],
        out_specs=pl.BlockSpec((1, Lq, d), lambda b, t: (b, 0, 0)),
        out_shape=jax.ShapeDtypeStruct((B, Lq, d), jnp.float32),
        scratch_shapes=[pltpu.VMEM((Lq, 256), jnp.float32)],
        compiler_params=pltpu.CompilerParams(
            dimension_semantics=("parallel", "arbitrary"),
        ),
    )(q, k, v)
